# baseline (speedup 1.0000x reference)
import jax
import jax.numpy as jnp
from jax import lax
from jax.experimental import pallas as pl
from jax.experimental.pallas import tpu as pltpu
from jax.experimental.pallas import tpu_sc as plsc

N = 100000
C = 64
NC, NS, L = 2, 16, 16
NW = NC * NS
CH = 512
FULL_CHUNKS = N // CH
TAIL_ROWS = N - FULL_CHUNKS * CH
TAIL_WORKER = FULL_CHUNKS % NW
MAX_CHUNKS_PER_W = (FULL_CHUNKS + NW - 1) // NW
BUF_WORDS = CH * C

_mesh = plsc.VectorSubcoreMesh(core_axis_name="c", subcore_axis_name="s")


@jax.jit
def _one_hot_sc(idx):
    @pl.kernel(
        out_type=jax.ShapeDtypeStruct((N, C), jnp.float32),
        mesh=_mesh,
        scratch_types=[
            pltpu.VMEM((CH, C), jnp.float32),
            pltpu.VMEM((CH,), jnp.int32),
        ],
        compiler_params=pltpu.CompilerParams(
            needs_layout_passes=False, use_tc_tiling_on_sc=True
        ),
    )
    def k(idx_hbm, out_hbm, buf, idx_v):
        wid = lax.axis_index("s") * NC + lax.axis_index("c")
        rvec = lax.iota(jnp.int32, L)
        ones = jnp.full((L,), 1.0, jnp.float32)
        zeros = jnp.zeros((L,), jnp.float32)

        def zinit(j, carry):
            for q in range(C // L):
                buf[j, pl.ds(q * L, L)] = zeros
            return carry

        lax.fori_loop(0, CH, zinit, 0)

        def scatter_chunk(n_groups, val):
            for g in range(n_groups):
                iv = idx_v[pl.ds(g * L, L)]
                plsc.store_scatter(buf, [rvec + g * L, iv], val)

        for i in range(MAX_CHUNKS_PER_W):
            c = wid + i * NW

            @pl.when(c < FULL_CHUNKS)
            def _():
                row0 = c * CH
                pltpu.sync_copy(idx_hbm.at[pl.ds(row0, CH)], idx_v)
                scatter_chunk(CH // L, ones)
                pltpu.sync_copy(buf, out_hbm.at[pl.ds(row0, CH)])
                scatter_chunk(CH // L, zeros)

        @pl.when(wid == TAIL_WORKER)
        def _():
            row0 = FULL_CHUNKS * CH
            pltpu.sync_copy(
                idx_hbm.at[pl.ds(row0, TAIL_ROWS)], idx_v.at[pl.ds(0, TAIL_ROWS)]
            )
            scatter_chunk(TAIL_ROWS // L, ones)
            pltpu.sync_copy(
                buf.at[pl.ds(0, TAIL_ROWS)],
                out_hbm.at[pl.ds(row0, TAIL_ROWS)],
            )

    return k(idx)


def kernel(species_index, pos):
    idx = species_index.astype(jnp.int32)
    one_hot = _one_hot_sc(idx).astype(pos.dtype)
    return (one_hot, one_hot)


# speedup vs baseline: 4.0607x; 4.0607x over previous
import jax
import jax.numpy as jnp
from jax import lax
from jax.experimental import pallas as pl
from jax.experimental.pallas import tpu as pltpu

N = 100000
C = 64
BI = 4096
GRID = (N + BI - 1) // BI


@jax.jit
def _tc_onehot_t(idx2d):
    def body(idx_ref, o1_ref, o2_ref):
        cls = lax.broadcasted_iota(jnp.int32, (C, BI), 0)
        oh = (cls == idx_ref[...]).astype(jnp.float32)
        o1_ref[...] = oh
        o2_ref[...] = oh

    s = jax.ShapeDtypeStruct((C, N), jnp.float32)
    return pl.pallas_call(
        body,
        grid=(GRID,),
        in_specs=[pl.BlockSpec((1, BI), lambda b: (0, b))],
        out_specs=[
            pl.BlockSpec((C, BI), lambda b: (0, b)),
            pl.BlockSpec((C, BI), lambda b: (0, b)),
        ],
        out_shape=[s, s],
    )(idx2d)


def kernel(species_index, pos):
    idx2d = species_index.astype(jnp.int32).reshape(1, N)
    o1t, o2t = _tc_onehot_t(idx2d)
    return (o1t.T.astype(pos.dtype), o2t.T.astype(pos.dtype))
